# Initial kernel scaffold; baseline (speedup 1.0000x reference)
import jax
import jax.numpy as jnp
from jax.experimental import pallas as pl


def kernel(node_ids, spatial_coords, categories, node_table, cat_table, W1, b1, W2, b2, Wf1, bf1, Wf2, bf2):
    raise NotImplementedError("write your pallas kernel here")


# speedup vs baseline: 10.9599x; 10.9599x over previous
import functools

import jax
import jax.numpy as jnp
from jax import lax
from jax.experimental import pallas as pl
from jax.experimental.pallas import tpu as pltpu
from jax.experimental.pallas import tpu_sc as plsc


def _make_sc_gather(N, D, chunk):
    info = plsc.get_sparse_core_info()
    NC, NS = info.num_cores, info.num_subcores
    NW = NC * NS
    b_per_w = N // NW
    n_chunks = b_per_w // chunk
    mesh = plsc.VectorSubcoreMesh(core_axis_name="c", subcore_axis_name="s")

    @functools.partial(
        pl.kernel,
        mesh=mesh,
        out_type=jax.ShapeDtypeStruct((N, D), jnp.float32),
        scratch_types=[
            pltpu.VMEM((chunk,), jnp.int32),
            pltpu.VMEM((chunk, D), jnp.float32),
            pltpu.SemaphoreType.DMA,
        ],
        compiler_params=pltpu.CompilerParams(use_tc_tiling_on_sc=False),
    )
    def gather_kernel(table_hbm, idx_hbm, out_hbm, idx_v, rows_v, sem):
        wid = lax.axis_index("s") * NC + lax.axis_index("c")
        base = wid * b_per_w

        def body(j, carry):
            off = pl.multiple_of(base + j * chunk, chunk)
            pltpu.sync_copy(idx_hbm.at[pl.ds(off, chunk)], idx_v)
            pltpu.async_copy(table_hbm.at[idx_v], rows_v, sem).wait()
            pltpu.sync_copy(rows_v, out_hbm.at[pl.ds(off, chunk)])
            return carry

        lax.fori_loop(0, n_chunks, body, 0)

    return gather_kernel


def _dense_body(g4_ref, side_ref, w1b_ref, b1b_ref, mb_ref, ctb_ref,
                bab_ref, wnb_ref, w2b_ref, bf2b_ref, o_ref):
    R = g4_ref.shape[0]
    g4 = g4_ref[...]
    side = side_ref[...]
    coords = side[:, 0:8]
    hp = jnp.maximum(
        jnp.dot(coords, w1b_ref[...], preferred_element_type=jnp.float32)
        + b1b_ref[...], 0.0)
    patt = (lax.broadcasted_iota(jnp.int32, (R, 32), 1) & 7).astype(jnp.float32)
    ohp = (side[:, 8:40] == patt).astype(jnp.float32)
    s4 = (jnp.dot(hp, mb_ref[...], preferred_element_type=jnp.float32)
          + jnp.dot(ohp, ctb_ref[...], preferred_element_type=jnp.float32)
          + bab_ref[...])
    a4 = jnp.maximum(
        jnp.dot(g4, wnb_ref[...], preferred_element_type=jnp.float32) + s4,
        0.0)
    o_ref[...] = (jnp.dot(a4, w2b_ref[...], preferred_element_type=jnp.float32)
                  + bf2b_ref[...])


def kernel(node_ids, spatial_coords, categories, node_table, cat_table,
           W1, b1, W2, b2, Wf1, bf1, Wf2, bf2):
    B, L = node_ids.shape
    V, D = node_table.shape
    N = B * L
    f32 = jnp.float32

    ids = node_ids.reshape(N)
    cats_rep = jnp.repeat(
        categories.reshape(N // 4, 4).astype(f32), 8, axis=1)
    side = jnp.concatenate(
        [spatial_coords.reshape(N // 4, 8), cats_rep], axis=1)

    W1t = W1.T
    Wnt = Wf1[:, :D].T
    Wsp = Wf1[:, D:D + D // 4]
    Mt = W2.T @ Wsp.T
    bias_a = (bf1 + Wsp @ b2).reshape(1, D)
    Ct = cat_table @ Wf1[:, D + D // 4:].T
    Ctp = jnp.concatenate([Ct, jnp.zeros((1, D), f32)], axis=0)
    Wf2t = Wf2.T

    def blkdiag4(m):
        k, n = m.shape
        out = jnp.zeros((4 * k, 4 * n), f32)
        for s in range(4):
            out = out.at[s * k:(s + 1) * k, s * n:(s + 1) * n].set(m)
        return out

    W1blk = blkdiag4(W1t)
    b1blk = jnp.tile(b1.reshape(1, -1), (1, 4))
    Mblk = blkdiag4(Mt)
    Ctblk = blkdiag4(Ctp)
    bab = jnp.tile(bias_a, (1, 4))
    Wnblk = blkdiag4(Wnt)
    Wf2blk = blkdiag4(Wf2t)
    bf2blk = jnp.tile(bf2.reshape(1, D), (1, 4))

    g = _make_sc_gather(N, D, 128)(node_table, ids)
    g4 = g.reshape(N // 4, 128)

    R = 1024
    NG = (N // 4) // R

    out4 = pl.pallas_call(
        _dense_body,
        grid=(NG,),
        in_specs=[
            pl.BlockSpec((R, 128), lambda i: (i, 0)),
            pl.BlockSpec((R, 40), lambda i: (i, 0)),
            pl.BlockSpec((8, 32), lambda i: (0, 0)),
            pl.BlockSpec((1, 32), lambda i: (0, 0)),
            pl.BlockSpec((32, 128), lambda i: (0, 0)),
            pl.BlockSpec((32, 128), lambda i: (0, 0)),
            pl.BlockSpec((1, 128), lambda i: (0, 0)),
            pl.BlockSpec((128, 128), lambda i: (0, 0)),
            pl.BlockSpec((128, 128), lambda i: (0, 0)),
            pl.BlockSpec((1, 128), lambda i: (0, 0)),
        ],
        out_specs=pl.BlockSpec((R, 128), lambda i: (i, 0)),
        out_shape=jax.ShapeDtypeStruct((N // 4, 128), f32),
    )(g4, side, W1blk, b1blk, Mblk, Ctblk, bab, Wnblk, Wf2blk, bf2blk)

    return out4.reshape(B, L, D)
